# Initial kernel scaffold; baseline (speedup 1.0000x reference)
import jax
import jax.numpy as jnp
from jax.experimental import pallas as pl


def kernel(x, ei_nbhd, ew_nbhd, ei_simi, ew_simi, ei_cont, ew_cont, params):
    raise NotImplementedError("write your pallas kernel here")


# speedup vs baseline: 24.8759x; 24.8759x over previous
import functools

import jax
import jax.numpy as jnp
from jax import lax
from jax.experimental import pallas as pl
from jax.experimental.pallas import tpu as pltpu
from jax.experimental.pallas import tpu_sc as plsc

_B, _T, _N, _F, _H = 2, 8, 10000, 128, 64
_E = 320000
_BT = _B * _T
_BH = _B * _H

_NC, _NS = 2, 16
_NW = _NC * _NS
_EPW = _E // _NW
_C = 80
_NCH = _EPW // _C
_RPT = _N // _NS
_EPS = _E // _NS
_NCHB = _EPS // _C
_NT = 400
_NG = _N // _NT

def _lane_select(ew16, e):
    return jnp.where(lax.iota(jnp.int32, 16) == e, ew16, 0.0)


@functools.lru_cache(maxsize=None)
def _build_sc_edge_weight_sums():
    mesh = plsc.VectorSubcoreMesh(core_axis_name="c", subcore_axis_name="s")

    @functools.partial(
        pl.kernel,
        out_type=jax.ShapeDtypeStruct((3, _NC, _NS, 5, 125, 16), jnp.float32),
        mesh=mesh,
        scratch_types=[
            pltpu.VMEM((_NCH, _C), jnp.int32),
            pltpu.VMEM((_NCH, _C), jnp.float32),
            pltpu.VMEM((_C, 16), jnp.float32),
            pltpu.VMEM((125, 16), jnp.float32),
            pltpu.VMEM_SHARED((_N, 16), jnp.float32),
        ],
        compiler_params=pltpu.CompilerParams(use_tc_tiling_on_sc=False),
    )
    def body(src_hbm, ew_hbm, zero_hbm, out_hbm,
             src_v, ew_v, rows_v, zbuf_v, acc_sh):
        cid = lax.axis_index("c")
        sid = lax.axis_index("s")
        wid = cid * _NS + sid
        pltpu.sync_copy(zero_hbm, zbuf_v)
        for g in range(3):
            for z in range(_RPT // 125):
                pltpu.sync_copy(
                    zbuf_v, acc_sh.at[pl.ds(sid * _RPT + z * 125, 125)])
            plsc.subcore_barrier()
            pltpu.sync_copy(src_hbm.at[g, wid], src_v)
            pltpu.sync_copy(ew_hbm.at[g, wid], ew_v)

            def chunk(c, carry):
                for gg in range(_C // 16):
                    ew16 = ew_v[c, pl.ds(gg * 16, 16)]
                    for e in range(16):
                        rows_v[gg * 16 + e, :] = _lane_select(ew16, e)
                pltpu.sync_copy(rows_v, acc_sh.at[src_v.at[c]], add=True)
                return carry

            lax.fori_loop(0, _NCH, chunk, 0)
            plsc.subcore_barrier()
            for z in range(_RPT // 125):
                pltpu.sync_copy(
                    acc_sh.at[pl.ds(sid * _RPT + z * 125, 125)], zbuf_v)
                pltpu.sync_copy(zbuf_v, out_hbm.at[g, cid, sid, z])
            pltpu.sync_copy(zero_hbm, zbuf_v)
            plsc.subcore_barrier()

    return body


def _sc_edge_weight_sums(src3, ew3, zeros_a):
    return _build_sc_edge_weight_sums()(src3, ew3, zeros_a)


@functools.lru_cache(maxsize=None)
def _build_sc_spmm():
    mesh = plsc.VectorSubcoreMesh(core_axis_name="c", subcore_axis_name="s")

    @functools.partial(
        pl.kernel,
        out_type=jax.ShapeDtypeStruct((3, _NC, _NS, 5, 125, _H), jnp.float32),
        mesh=mesh,
        scratch_types=[
            pltpu.VMEM((_NCHB, _C), jnp.int32),
            pltpu.VMEM((_NCHB, _C), jnp.int32),
            pltpu.VMEM((_NCHB, _C), jnp.float32),
            pltpu.VMEM((_C, _H), jnp.float32),
            pltpu.VMEM((125, _H), jnp.float32),
            pltpu.VMEM((125, _H), jnp.float32),
            pltpu.VMEM_SHARED((_N, _H), jnp.float32),
            pltpu.SemaphoreType.DMA,
        ],
        compiler_params=pltpu.CompilerParams(use_tc_tiling_on_sc=False),
    )
    def body(hm0, hm1, hm2, src_hbm, dst_hbm, ew_hbm, zero_hbm, out_hbm,
             src_v, dst_v, ew_v, rows_v, stage_v, zbuf_v, acc_sh, sem):
        cid = lax.axis_index("c")
        sid = lax.axis_index("s")
        pltpu.sync_copy(zero_hbm, zbuf_v)
        for g, hm in enumerate((hm0, hm1, hm2)):
            for z in range(_RPT // 125):
                pltpu.sync_copy(
                    zbuf_v, acc_sh.at[pl.ds(sid * _RPT + z * 125, 125)])
            plsc.subcore_barrier()
            pltpu.sync_copy(src_hbm.at[g, sid], src_v)
            pltpu.sync_copy(dst_hbm.at[g, sid], dst_v)
            pltpu.sync_copy(ew_hbm.at[g, sid], ew_v)

            def chunk(c, carry, hm=hm):
                pltpu.async_copy(hm.at[cid].at[src_v.at[c]], rows_v,
                                 sem).wait()
                for gg in range(_C // 16):
                    ew16 = ew_v[c, pl.ds(gg * 16, 16)]
                    for e in range(16):
                        w = ew16[e]
                        r = gg * 16 + e
                        for j in range(_H // 16):
                            rows_v[r, pl.ds(j * 16, 16)] = (
                                rows_v[r, pl.ds(j * 16, 16)] * w)
                pltpu.sync_copy(rows_v, acc_sh.at[dst_v.at[c]], add=True)
                return carry

            lax.fori_loop(0, _NCHB, chunk, 0)
            plsc.subcore_barrier()
            for z in range(_RPT // 125):
                pltpu.sync_copy(
                    acc_sh.at[pl.ds(sid * _RPT + z * 125, 125)], stage_v)
                pltpu.sync_copy(stage_v, out_hbm.at[g, cid, sid, z])
            plsc.subcore_barrier()

    return body


def _sc_spmm(hm0, hm1, hm2, srcB, dstB, ewB, zeros_b):
    return _build_sc_spmm()(hm0, hm1, hm2, srcB, dstB, ewB, zeros_b)


def _tc1_body(x_ref, wt_ref, wp_ref, xw_ref, xsum_ref, wx_ref):
    i = pl.program_id(0)
    xb = x_ref[...]
    x2 = xb.reshape(_BT * _NT, _F)
    xw = jnp.dot(x2, wt_ref[...], preferred_element_type=jnp.float32)
    xw_ref[...] = xw.reshape(_B, _T, _NT, _H)
    w3 = wp_ref[...].sum(axis=(1, 3))
    x3 = xb.reshape(_BT, _NT, _F)
    ps = xb.sum(axis=2)
    pw = jnp.stack(
        [jnp.dot(w3, x3[bt], preferred_element_type=jnp.float32)
         for bt in range(_BT)], axis=1)

    @pl.when(i == 0)
    def _():
        xsum_ref[...] = ps
        wx_ref[...] = pw

    @pl.when(i > 0)
    def _():
        xsum_ref[...] += ps
        wx_ref[...] += pw


def _tc_pass1(x, wt, wpart):
    return pl.pallas_call(
        _tc1_body,
        grid=(_NG,),
        in_specs=[
            pl.BlockSpec((_B, _T, _NT, _F), lambda i: (0, 0, i, 0)),
            pl.BlockSpec((_F, _H), lambda i: (0, 0)),
            pl.BlockSpec((3, _NC, _NT, 16), lambda i: (0, 0, i, 0)),
        ],
        out_specs=[
            pl.BlockSpec((_B, _T, _NT, _H), lambda i: (0, 0, i, 0)),
            pl.BlockSpec((_B, _T, _F), lambda i: (0, 0, 0)),
            pl.BlockSpec((3, _BT, _F), lambda i: (0, 0, 0)),
        ],
        out_shape=[
            jax.ShapeDtypeStruct((_B, _T, _N, _H), jnp.float32),
            jax.ShapeDtypeStruct((_B, _T, _F), jnp.float32),
            jax.ShapeDtypeStruct((3, _BT, _F), jnp.float32),
        ],
    )(x, wt, wpart)


def _elu(v):
    return jnp.where(v > 0, v, jnp.exp(v) - 1.0)


def _tcg_body(xsum_ref, wx_ref, w1_ref, b1_ref, w2_ref, b2_ref, g_ref):
    xs = xsum_ref[...].reshape(_BT, _F) * (1.0 / _N)
    for br in range(3):
        wx = wx_ref[br] * (1.0 / _N)
        ap = jnp.concatenate([xs, wx], axis=-1)
        h = _elu(jnp.dot(ap, w1_ref[br],
                         preferred_element_type=jnp.float32) + b1_ref[br])
        z = (h * w2_ref[br][None, :]).sum(axis=-1) + b2_ref[br, 0]
        g_ref[br, :] = 1.0 / (1.0 + jnp.exp(-z))


def _tc_gate(xsum, wx, w1s, b1s, w2s, b2s):
    return pl.pallas_call(
        _tcg_body,
        in_specs=[
            pl.BlockSpec((_B, _T, _F), lambda: (0, 0, 0)),
            pl.BlockSpec((3, _BT, _F), lambda: (0, 0, 0)),
            pl.BlockSpec((3, 2 * _F, _F), lambda: (0, 0, 0)),
            pl.BlockSpec((3, _F), lambda: (0, 0)),
            pl.BlockSpec((3, _F), lambda: (0, 0)),
            pl.BlockSpec((3, 1), lambda: (0, 0), memory_space=pltpu.SMEM),
        ],
        out_specs=pl.BlockSpec((3, _BT), lambda: (0, 0)),
        out_shape=jax.ShapeDtypeStruct((3, _BT), jnp.float32),
    )(xsum, wx, w1s, b1s, w2s, b2s)


def _tc2_body(xw_ref, g_ref, bt_ref, hm_ref):
    btv = bt_ref[0, :]
    for br in range(3):
        for b in range(_B):
            acc = jnp.zeros((_NT, _H), jnp.float32)
            for t in range(_T):
                acc += _elu(g_ref[br, b * _T + t] * xw_ref[b, t]
                            + btv[None, :])
            hm_ref[br, b] = acc * (1.0 / _T)


def _tc_pass2(xw, g3, bt):
    return pl.pallas_call(
        _tc2_body,
        grid=(_NG,),
        in_specs=[
            pl.BlockSpec((_B, _T, _NT, _H), lambda i: (0, 0, i, 0)),
            pl.BlockSpec((3, _BT), lambda i: (0, 0), memory_space=pltpu.SMEM),
            pl.BlockSpec((1, _H), lambda i: (0, 0)),
        ],
        out_specs=pl.BlockSpec((3, _B, _NT, _H), lambda i: (0, 0, i, 0)),
        out_shape=jax.ShapeDtypeStruct((3, _B, _N, _H), jnp.float32),
    )(xw, g3, bt)


def _tc3_body(hm_ref, lh_ref, w0_ref, w1_ref, fcw_ref, fcb_ref, out_ref):
    fcw = fcw_ref[0, :]
    outs = {}
    for br in range(3):
        for b in range(_B):
            hnb = hm_ref[br, b]
            lnb = lh_ref[br, b]
            o = jnp.dot(hnb, w0_ref[br], preferred_element_type=jnp.float32)
            o += jnp.dot(lnb, w1_ref[br], preferred_element_type=jnp.float32)
            outs[(b, br)] = jnp.maximum(o, 0.0)
    cols = []
    for b in range(_B):
        cat = jnp.concatenate([outs[(b, 0)], outs[(b, 1)], outs[(b, 2)]],
                              axis=-1)
        v = (cat * fcw[None, :]).sum(axis=-1) + fcb_ref[0, 0]
        cols.append(jnp.maximum(v, 0.0)[:, None])
    out_ref[...] = jnp.concatenate(cols, axis=-1)


def _tc_pass3(hm3, lhp, w0s, w1s, fcw, fcb):
    return pl.pallas_call(
        _tc3_body,
        grid=(_NG,),
        in_specs=[
            pl.BlockSpec((3, _B, _NT, _H), lambda i: (0, 0, i, 0)),
            pl.BlockSpec((3, _B, _NT, _H), lambda i: (0, 0, i, 0)),
            pl.BlockSpec((3, _H, _H), lambda i: (0, 0, 0)),
            pl.BlockSpec((3, _H, _H), lambda i: (0, 0, 0)),
            pl.BlockSpec((1, 3 * _H), lambda i: (0, 0)),
            pl.BlockSpec((1, 1), lambda i: (0, 0), memory_space=pltpu.SMEM),
        ],
        out_specs=pl.BlockSpec((_NT, _B), lambda i: (i, 0)),
        out_shape=jax.ShapeDtypeStruct((_N, _B), jnp.float32),
    )(hm3, lhp, w0s, w1s, fcw, fcb)


def kernel(x, ei_nbhd, ew_nbhd, ei_simi, ew_simi, ei_cont, ew_cont, params):
    p = params
    src3 = jnp.stack([ei_nbhd[0], ei_simi[0], ei_cont[0]])
    dst3 = jnp.stack([ei_nbhd[1], ei_simi[1], ei_cont[1]])
    ew3 = jnp.stack([ew_nbhd, ew_simi, ew_cont])
    srcA = src3.reshape(3, _NW, _NCH, _C)
    ewA = ew3.reshape(3, _NW, _NCH, _C)
    srcB = src3.reshape(3, _NS, _NCHB, _C)
    dstB = dst3.reshape(3, _NS, _NCHB, _C)
    ewB = ew3.reshape(3, _NS, _NCHB, _C)

    zeros_a = jnp.zeros((125, 16), jnp.float32)
    zeros_b = jnp.zeros((125, _H), jnp.float32)

    wpart = _sc_edge_weight_sums(srcA, ewA, zeros_a
                                 ).reshape(3, _NC, _N, 16)

    xw, xsum, wx = _tc_pass1(x, p['temp_W'], wpart)

    w1s = jnp.stack([p['gate_W1_nbhd'], p['gate_W1_simi'], p['gate_W1_cont']])
    b1s = jnp.stack([p['gate_b1_nbhd'], p['gate_b1_simi'], p['gate_b1_cont']])
    w2s = jnp.stack([p['gate_W2_nbhd'], p['gate_W2_simi'], p['gate_W2_cont']]
                    ).reshape(3, _F)
    b2s = jnp.stack([p['gate_b2_nbhd'], p['gate_b2_simi'], p['gate_b2_cont']]
                    ).reshape(3, 1)
    g3 = _tc_gate(xsum, wx, w1s, b1s, w2s, b2s)

    hm4 = _tc_pass2(xw, g3, p['temp_b'].reshape(1, _H))

    lhp = _sc_spmm(hm4[0], hm4[1], hm4[2], srcB, dstB, ewB, zeros_b
                   ).reshape(3, _B, _N, _H)

    w0s = jnp.stack([p['mg_W0_nbhd'], p['mg_W0_simi'], p['mg_W0_cont']])
    w1g = jnp.stack([p['mg_W1_nbhd'], p['mg_W1_simi'], p['mg_W1_cont']])
    fcw = p['fc_W'].reshape(1, 3 * _H)
    fcb = p['fc_b'].reshape(1, 1)
    return _tc_pass3(hm4, lhp, w0s, w1g, fcw, fcb).T
